# Initial kernel scaffold; baseline (speedup 1.0000x reference)
import jax
import jax.numpy as jnp
from jax.experimental import pallas as pl


def kernel(x, table):
    raise NotImplementedError("write your pallas kernel here")


# speedup vs baseline: 1.0941x; 1.0941x over previous
import functools

import jax
import jax.numpy as jnp
from jax import lax
from jax.experimental import pallas as pl
from jax.experimental.pallas import tpu as pltpu
from jax.experimental.pallas import tpu_sc as plsc

_VOCAB = 1000000
_EMB = 32
_BATCH = 16384
_HIST = 50
_N = _BATCH * _HIST

_NC = 2
_NS = 16
_NW = _NC * _NS
_B_PER_W = _N // _NW
_CHUNK = 1024
_NCHUNK = _B_PER_W // _CHUNK


def _gather_kernel(idx_hbm, table_hbm, out_hbm, idx_v, rows_v, sem):
    wid = lax.axis_index("s") * _NC + lax.axis_index("c")
    base = wid * _B_PER_W

    def body(c, carry):
        off = base + c * _CHUNK
        pltpu.sync_copy(idx_hbm.at[pl.ds(off, _CHUNK)], idx_v)
        pltpu.async_copy(table_hbm.at[idx_v], rows_v, sem).wait()
        pltpu.sync_copy(rows_v, out_hbm.at[pl.ds(off, _CHUNK)])
        return carry

    lax.fori_loop(0, _NCHUNK, body, 0)


@functools.partial(
    pl.kernel,
    mesh=plsc.VectorSubcoreMesh(core_axis_name="c", subcore_axis_name="s"),
    compiler_params=pltpu.CompilerParams(use_tc_tiling_on_sc=False),
    out_type=jax.ShapeDtypeStruct((_N, _EMB), jnp.float32),
    scratch_types=[
        pltpu.VMEM((_CHUNK,), jnp.int32),
        pltpu.VMEM((_CHUNK, _EMB), jnp.float32),
        pltpu.SemaphoreType.DMA,
    ],
)
def _embed_lookup(idx_hbm, table_hbm, out_hbm, idx_v, rows_v, sem):
    _gather_kernel(idx_hbm, table_hbm, out_hbm, idx_v, rows_v, sem)


def kernel(x, table):
    idx = x.reshape(_N).astype(jnp.int32)
    out = _embed_lookup(idx, table)
    return out.reshape(_BATCH, _HIST, _EMB)


# speedup vs baseline: 1.1121x; 1.0164x over previous
import functools

import jax
import jax.numpy as jnp
from jax import lax
from jax.experimental import pallas as pl
from jax.experimental.pallas import tpu as pltpu
from jax.experimental.pallas import tpu_sc as plsc

_VOCAB = 1000000
_EMB = 32
_BATCH = 16384
_HIST = 50
_N = _BATCH * _HIST

_NC = 2
_NS = 16
_NW = _NC * _NS
_B_PER_W = _N // _NW
_CHUNK = 1600
_NCHUNK = _B_PER_W // _CHUNK


def _body(idx_hbm, table_hbm, out_hbm, idx_v, rows_v,
          si0, si1, sg0, sg1, ss0, ss1):
    wid = lax.axis_index("s") * _NC + lax.axis_index("c")
    base = wid * _B_PER_W
    sem_i = (si0, si1)
    sem_g = (sg0, sg1)
    sem_s = (ss0, ss1)

    def idx_load(c):
        b = c % 2
        return pltpu.async_copy(
            idx_hbm.at[pl.ds(base + c * _CHUNK, _CHUNK)],
            idx_v.at[b], sem_i[b])

    def gather(c):
        b = c % 2
        return pltpu.async_copy(table_hbm.at[idx_v.at[b]], rows_v.at[b],
                                sem_g[b])

    def store(c):
        b = c % 2
        return pltpu.async_copy(
            rows_v.at[b],
            out_hbm.at[pl.ds(base + c * _CHUNK, _CHUNK)], sem_s[b])

    il = {0: idx_load(0), 1: idx_load(1)}
    gt = {}
    st = {}
    for c in range(_NCHUNK):
        b = c % 2
        if c >= 2:
            st[c - 2].wait()
        il[c].wait()
        gt[c] = gather(c)
        if c >= 1:
            gt[c - 1].wait()
            st[c - 1] = store(c - 1)
            if c + 1 < _NCHUNK:
                il[c + 1] = idx_load(c + 1)
    gt[_NCHUNK - 1].wait()
    st[_NCHUNK - 1] = store(_NCHUNK - 1)
    st[_NCHUNK - 2].wait()
    st[_NCHUNK - 1].wait()


@functools.partial(
    pl.kernel,
    mesh=plsc.VectorSubcoreMesh(core_axis_name="c", subcore_axis_name="s"),
    compiler_params=pltpu.CompilerParams(use_tc_tiling_on_sc=False),
    out_type=jax.ShapeDtypeStruct((_N, _EMB), jnp.float32),
    scratch_types=[
        pltpu.VMEM((2, _CHUNK), jnp.int32),
        pltpu.VMEM((2, _CHUNK, _EMB), jnp.float32),
        pltpu.SemaphoreType.DMA,
        pltpu.SemaphoreType.DMA,
        pltpu.SemaphoreType.DMA,
        pltpu.SemaphoreType.DMA,
        pltpu.SemaphoreType.DMA,
        pltpu.SemaphoreType.DMA,
    ],
)
def _embed_lookup(idx_hbm, table_hbm, out_hbm, idx_v, rows_v,
                  si0, si1, sg0, sg1, ss0, ss1):
    _body(idx_hbm, table_hbm, out_hbm, idx_v, rows_v,
          si0, si1, sg0, sg1, ss0, ss1)


def kernel(x, table):
    idx = x.reshape(_N).astype(jnp.int32)
    out = _embed_lookup(idx, table)
    return out.reshape(_BATCH, _HIST, _EMB)
